# baseline (speedup 1.0000x reference)
import functools
import jax
import jax.numpy as jnp
from jax import lax
from jax.experimental import pallas as pl
from jax.experimental.pallas import tpu as pltpu
from jax.experimental.pallas import tpu_sc as plsc

_B, _LEN, _D, _V = 4096, 200, 64, 1000000
_NC, _NS = 2, 16
_NW = _NC * _NS
_NBLK = _V // 128
_BPW_T = _NBLK // _NW
_XTRA = _NBLK - _BPW_T * _NW


def _sig_body(y_ref, w_ref, b_ref, o_ref):
    o_ref[...] = (
        jnp.dot(w_ref[...], y_ref[...].T, preferred_element_type=jnp.float32)
        + b_ref[...]
    )


def _compute_sig_t(y, w, bias):
    return pl.pallas_call(
        _sig_body,
        out_shape=jax.ShapeDtypeStruct((_D, _B), jnp.float32),
    )(y, w, bias.reshape(_D, 1))


@functools.partial(
    pl.kernel,
    out_type=jax.ShapeDtypeStruct((_V // 2, 128), jnp.float32),
    mesh=plsc.VectorSubcoreMesh(
        core_axis_name="c", subcore_axis_name="s", num_cores=_NC, num_subcores=_NS
    ),
    scratch_types=[
        pltpu.VMEM((_D, 129), jnp.float32),
        pltpu.VMEM((_D, 129), jnp.float32),
        pltpu.VMEM((_D, 128), jnp.float32),
        pltpu.VMEM((_D, 128), jnp.float32),
        pltpu.SemaphoreType.DMA,
        pltpu.SemaphoreType.DMA,
        pltpu.SemaphoreType.DMA,
        pltpu.SemaphoreType.DMA,
    ],
    compiler_params=pltpu.CompilerParams(use_tc_tiling_on_sc=True, needs_layout_passes=False),
)
def _sc_transpose(
    embt_hbm, tail_hbm, out_hbm,
    tb0, tb1, wb0, wb1, rsem0, rsem1, wsem0, wsem1,
):
    wid = lax.axis_index("s") * _NC + lax.axis_index("c")
    base = wid * _BPW_T + jnp.minimum(wid, _XTRA)

    iotas = [lax.iota(jnp.int32, 16) + 16 * k for k in range(_D // 16)]

    def fire_read(t, tb, rsem):
        pltpu.async_copy(
            embt_hbm.at[:, pl.ds(t * 128, 128)], tb.at[:, pl.ds(0, 128)], rsem
        )

    def wait_read(tb, rsem):
        pltpu.make_async_copy(
            embt_hbm.at[:, pl.ds(0, 128)], tb.at[:, pl.ds(0, 128)], rsem
        ).wait()

    def fire_write(t, wb, wsem):
        pltpu.async_copy(wb, out_hbm.at[pl.ds(t * _D, _D)], wsem)

    def wait_write(wb, wsem):
        pltpu.make_async_copy(wb, out_hbm.at[pl.ds(0, _D)], wsem).wait()

    def transpose(tb, wb):
        def grp_body(g, carry):
            j0 = g * 16
            for dj in range(16):
                j = j0 + dj
                jvec = jnp.full((16,), 0, jnp.int32) + j
                row = j >> 1
                cb = (dj & 1) * _D
                for k in range(_D // 16):
                    col = plsc.load_gather(tb, [iotas[k], jvec])
                    wb[row, pl.ds(cb + 16 * k, 16)] = col
            return carry

        lax.fori_loop(0, 8, grp_body, 0)

    fire_read(base, tb0, rsem0)

    def pair_body(k, carry):
        t0 = base + 2 * k
        t1 = t0 + 1

        fire_read(t1, tb1, rsem1)
        wait_read(tb0, rsem0)

        @pl.when(k > 0)
        def _():
            wait_write(wb0, wsem0)
        transpose(tb0, wb0)
        fire_write(t0, wb0, wsem0)

        @pl.when(k < _BPW_T // 2 - 1)
        def _():
            fire_read(t0 + 2, tb0, rsem0)
        wait_read(tb1, rsem1)

        @pl.when(k > 0)
        def _():
            wait_write(wb1, wsem1)
        transpose(tb1, wb1)
        fire_write(t1, wb1, wsem1)
        return carry

    lax.fori_loop(0, _BPW_T // 2, pair_body, 0)
    wait_write(wb0, wsem0)
    wait_write(wb1, wsem1)

    @pl.when(wid < _XTRA)
    def _():
        t = base + _BPW_T
        pltpu.sync_copy(
            embt_hbm.at[:, pl.ds(t * 128, 128)], tb0.at[:, pl.ds(0, 128)]
        )
        transpose(tb0, wb0)
        pltpu.sync_copy(wb0, out_hbm.at[pl.ds(t * _D, _D)])

    @pl.when(wid == _NW - 1)
    def _():
        pltpu.sync_copy(tail_hbm, out_hbm.at[pl.ds(_NBLK * _D, 32)])


@functools.partial(
    pl.kernel,
    out_type=jax.ShapeDtypeStruct((_LEN, _D, _B), jnp.float32),
    mesh=plsc.VectorSubcoreMesh(
        core_axis_name="c", subcore_axis_name="s", num_cores=_NC, num_subcores=_NS
    ),
    scratch_types=[
        pltpu.VMEM((_LEN, 128), jnp.int32),
        pltpu.VMEM((_D, 128), jnp.float32),
        pltpu.VMEM((128,), jnp.int32),
        pltpu.VMEM((128,), jnp.int32),
        pltpu.VMEM((128, 129), jnp.float32),
        pltpu.VMEM((128, 129), jnp.float32),
        pltpu.VMEM((_D, 128), jnp.float32),
        pltpu.VMEM((_D, 128), jnp.float32),
        pltpu.SemaphoreType.DMA,
        pltpu.SemaphoreType.DMA,
        pltpu.SemaphoreType.DMA,
        pltpu.SemaphoreType.DMA,
    ],
    compiler_params=pltpu.CompilerParams(use_tc_tiling_on_sc=True, needs_layout_passes=False),
)
def _sc_embed(
    xt_hbm, sigt_hbm, table_hbm, out_hbm,
    xfull, sig_v, jr0, jr1, pb0, pb1, ob0, ob1, gsem0, gsem1, osem0, osem1,
):
    wid = lax.axis_index("s") * _NC + lax.axis_index("c")
    b0 = wid * 128

    pltpu.sync_copy(xt_hbm.at[:, pl.ds(b0, 128)], xfull)
    pltpu.sync_copy(sigt_hbm.at[:, pl.ds(b0, 128)], sig_v)

    iotas = [lax.iota(jnp.int32, 16) + 16 * q for q in range(8)]

    def fill_indices(l, jr):
        for q in range(8):
            jr[pl.ds(16 * q, 16)] = xfull[l, pl.ds(16 * q, 16)] >> 1

    def fire_gather(jr, pb, gsem):
        pltpu.async_copy(table_hbm.at[jr], pb.at[:, pl.ds(0, 128)], gsem)

    def wait_gather(jr, pb, gsem):
        pltpu.make_async_copy(
            table_hbm.at[jr], pb.at[:, pl.ds(0, 128)], gsem
        ).wait()

    def fire_write(l, ob, osem):
        pltpu.async_copy(ob, out_hbm.at[l, :, pl.ds(b0, 128)], osem)

    def wait_write(ob, osem):
        pltpu.make_async_copy(ob, out_hbm.at[0, :, pl.ds(b0, 128)], osem).wait()

    def process(l, pb, ob):
        hcols = [(xfull[l, pl.ds(16 * q, 16)] & 1) << 6 for q in range(8)]

        def dgrp_body(g, carry):
            d0 = g * 8
            for dd in range(8):
                d = d0 + dd
                for q in range(8):
                    v = plsc.load_gather(pb, [iotas[q], hcols[q] + d])
                    ob[d, pl.ds(16 * q, 16)] = v + sig_v[d, pl.ds(16 * q, 16)]
            return carry

        lax.fori_loop(0, 8, dgrp_body, 0)

    fill_indices(0, jr0)
    fire_gather(jr0, pb0, gsem0)

    def pair_body(i, carry):
        l0 = 2 * i
        l1 = l0 + 1

        fill_indices(l1, jr1)
        fire_gather(jr1, pb1, gsem1)
        wait_gather(jr0, pb0, gsem0)

        @pl.when(i > 0)
        def _():
            wait_write(ob0, osem0)
        process(l0, pb0, ob0)
        fire_write(l0, ob0, osem0)

        @pl.when(i < _LEN // 2 - 1)
        def _():
            fill_indices(l1 + 1, jr0)
            fire_gather(jr0, pb0, gsem0)
        wait_gather(jr1, pb1, gsem1)

        @pl.when(i > 0)
        def _():
            wait_write(ob1, osem1)
        process(l1, pb1, ob1)
        fire_write(l1, ob1, osem1)
        return carry

    lax.fori_loop(0, _LEN // 2, pair_body, 0)
    wait_write(ob0, osem0)
    wait_write(ob1, osem1)


@jax.jit
def kernel(x, y, embedding, W, b):
    xt = x.T
    embt = embedding.T
    sigt = _compute_sig_t(y, W, b)
    tail = embedding[_V - 64:, :].reshape(32, 128)
    table2 = _sc_transpose(embt, tail)
    outt = _sc_embed(xt, sigt, table2)
    return jnp.transpose(outt, (2, 0, 1))


# speedup vs baseline: 2.1479x; 2.1479x over previous
import functools
import jax
import jax.numpy as jnp
from jax import lax
from jax.experimental import pallas as pl
from jax.experimental.pallas import tpu as pltpu
from jax.experimental.pallas import tpu_sc as plsc

_B, _LEN, _D, _V = 4096, 200, 64, 1000000
_NC, _NS = 2, 16
_NW = _NC * _NS
_BPW = _B // _NW
_RPW = _BPW * _LEN
_CH = _LEN
_UNITS = ((0, 104), (104, 96))
_NCHUNK = _RPW // _CH


def _sig_body(y_ref, w_ref, b_ref, o_ref):
    o_ref[...] = (
        jnp.dot(y_ref[...], w_ref[...].T, preferred_element_type=jnp.float32)
        + b_ref[...]
    )


def _compute_sig(y, w, bias):
    return pl.pallas_call(
        _sig_body,
        out_shape=jax.ShapeDtypeStruct((_B, _D), jnp.float32),
    )(y, w, bias.reshape(1, _D))


@functools.partial(
    pl.kernel,
    out_type=jax.ShapeDtypeStruct((_B, _LEN // 2, 128), jnp.float32),
    mesh=plsc.VectorSubcoreMesh(
        core_axis_name="c", subcore_axis_name="s", num_cores=_NC, num_subcores=_NS
    ),
    scratch_types=[
        pltpu.VMEM((_RPW,), jnp.int32),
        pltpu.VMEM((_BPW * _D,), jnp.float32),
        pltpu.VMEM((_CH, 128), jnp.float32),
        pltpu.VMEM((_CH, 128), jnp.float32),
        pltpu.VMEM((_CH // 2, 128), jnp.float32),
        pltpu.VMEM((_CH // 2, 128), jnp.float32),
        pltpu.SemaphoreType.DMA,
        pltpu.SemaphoreType.DMA,
        pltpu.SemaphoreType.DMA,
        pltpu.SemaphoreType.DMA,
    ],
    compiler_params=pltpu.CompilerParams(use_tc_tiling_on_sc=False),
)
def _sc_embed(
    idx_hbm, sig_hbm, table_hbm, out_hbm,
    idx_v, sig_v, buf0, buf1, wb0, wb1, gsem0, gsem1, osem0, osem1,
):
    wid = lax.axis_index("s") * _NC + lax.axis_index("c")
    base_b = wid * _BPW

    pltpu.sync_copy(idx_hbm.at[pl.ds(wid * _RPW, _RPW)], idx_v)
    pltpu.sync_copy(sig_hbm.at[pl.ds(wid * _BPW * _D, _BPW * _D)], sig_v)

    def fire_gather(c, buf, gsem):
        for off, n in _UNITS:
            pltpu.async_copy(
                table_hbm.at[idx_v.at[pl.ds(c * _CH + off, n)]],
                buf.at[pl.ds(off, n)],
                gsem,
            )

    def wait_gather(c, buf, gsem):
        for off, n in _UNITS:
            pltpu.make_async_copy(
                table_hbm.at[idx_v.at[pl.ds(c * _CH + off, n)]],
                buf.at[pl.ds(off, n)],
                gsem,
            ).wait()

    def fire_write(c, wb, osem):
        pltpu.async_copy(wb, out_hbm.at[base_b + c], osem)

    def wait_write(wb, osem):
        pltpu.make_async_copy(wb, out_hbm.at[base_b], osem).wait()

    def add_sig_pack(c, buf, wb):
        svs = [
            sig_v[pl.ds(c * _D + 16 * d, 16)] for d in range(_D // 16)
        ]

        def row_body(r2, carry):
            for d in range(_D // 16):
                wb[r2, pl.ds(16 * d, 16)] = (
                    buf[2 * r2, pl.ds(16 * d, 16)] + svs[d]
                )
                wb[r2, pl.ds(_D + 16 * d, 16)] = (
                    buf[2 * r2 + 1, pl.ds(16 * d, 16)] + svs[d]
                )
            return carry

        lax.fori_loop(0, _CH // 2, row_body, 0, unroll=4)

    fire_gather(0, buf0, gsem0)

    def pair_body(i, carry):
        c0 = 2 * i
        c1 = c0 + 1

        fire_gather(c1, buf1, gsem1)
        wait_gather(c0, buf0, gsem0)

        @pl.when(i > 0)
        def _():
            wait_write(wb0, osem0)
        add_sig_pack(c0, buf0, wb0)
        fire_write(c0, wb0, osem0)

        @pl.when(i < _NCHUNK // 2 - 1)
        def _():
            fire_gather(c0 + 2, buf0, gsem0)
        wait_gather(c1, buf1, gsem1)

        @pl.when(i > 0)
        def _():
            wait_write(wb1, osem1)
        add_sig_pack(c1, buf1, wb1)
        fire_write(c1, wb1, osem1)
        return carry

    lax.fori_loop(0, _NCHUNK // 2, pair_body, 0)

    wait_write(wb0, osem0)
    wait_write(wb1, osem1)


@jax.jit
def kernel(x, y, embedding, W, b):
    sig = _compute_sig(y, W, b)
    table2 = jnp.pad(embedding, ((0, 0), (0, 128 - _D)))
    out4 = _sc_embed(x.reshape(_B * _LEN), sig.reshape(_B * _D), table2)
    return out4.reshape(_B, _LEN, _D)
